# baseline (speedup 1.0000x reference)
import functools

import jax
import jax.numpy as jnp
from jax import lax
from jax.experimental import pallas as pl
from jax.experimental.pallas import tpu as pltpu
from jax.experimental.pallas import tpu_sc as plsc


def _build(B, D):
    info = plsc.get_sparse_core_info()
    NC, NS = info.num_cores, info.num_subcores
    NW = NC * NS
    b_per_w = B // NW
    CH = 128
    n_ch = b_per_w // CH

    mesh = plsc.VectorSubcoreMesh(core_axis_name="c", subcore_axis_name="s")

    @functools.partial(
        pl.kernel,
        mesh=mesh,
        out_type=jax.ShapeDtypeStruct((B, D), jnp.float32),
        scratch_types=[
            pltpu.VMEM((n_ch, CH), jnp.int32),
            pltpu.VMEM((b_per_w, D), jnp.float32),
            pltpu.SemaphoreType.DMA,
        ],
        compiler_params=pltpu.CompilerParams(use_tc_tiling_on_sc=False),
    )
    def gather_kernel(idx_hbm, table_hbm, out_hbm, idx_v, rows_v, sem):
        wid = lax.axis_index("s") * NC + lax.axis_index("c")
        base = wid * b_per_w
        pltpu.sync_copy(idx_hbm.at[wid], idx_v)
        copies = [
            pltpu.async_copy(
                table_hbm.at[idx_v.at[j]],
                rows_v.at[pl.ds(j * CH, CH)],
                sem,
            )
            for j in range(n_ch)
        ]
        for c in copies:
            c.wait()
        pltpu.sync_copy(rows_v, out_hbm.at[pl.ds(base, b_per_w)])

    return gather_kernel


def kernel(g, h, table):
    B = h.shape[0]
    D = table.shape[1]
    info = plsc.get_sparse_core_info()
    NW = info.num_cores * info.num_subcores
    CH = 128
    idx = h.astype(jnp.int32).reshape(NW, (B // NW) // CH, CH)
    return _build(B, D)(idx, table)


# speedup vs baseline: 5.2536x; 5.2536x over previous
import functools

import jax
import jax.numpy as jnp
from jax import lax
from jax.experimental import pallas as pl
from jax.experimental.pallas import tpu as pltpu
from jax.experimental.pallas import tpu_sc as plsc


def _build(B, V):
    info = plsc.get_sparse_core_info()
    NC, NS = info.num_cores, info.num_subcores
    NW = NC * NS
    b_per_w = B // NW
    CHUNK = 16
    n_chunks = b_per_w // CHUNK

    mesh = plsc.VectorSubcoreMesh(core_axis_name="c", subcore_axis_name="s")

    @functools.partial(
        pl.kernel,
        mesh=mesh,
        out_type=jax.ShapeDtypeStruct((2, 8, B), jnp.float32),
        scratch_types=[
            pltpu.VMEM((b_per_w,), jnp.int32),
            pltpu.VMEM((2, 8, CHUNK * 128), jnp.float32),
            pltpu.VMEM((2, 8, b_per_w), jnp.float32),
            pltpu.SemaphoreType.DMA,
        ],
        compiler_params=pltpu.CompilerParams(needs_layout_passes=False),
    )
    def gather_kernel(idx_hbm, tab_hbm, out_hbm, idx_v, buf_v, rows_v, sem):
        wid = lax.axis_index("s") * NC + lax.axis_index("c")
        base = wid * b_per_w
        lane = lax.iota(jnp.int32, 16)
        pltpu.sync_copy(idx_hbm.at[wid], idx_v)

        def chunk_body(chunk, _):
            c0 = chunk * CHUNK

            vec = idx_v[pl.ds(c0, 16)]
            cps = []
            for j in range(16):
                ralign = pl.multiple_of(
                    (vec[j] >> jnp.int32(7)) * jnp.int32(128), 128
                )
                cps.append(pltpu.async_copy(
                    tab_hbm.at[:, :, pl.ds(ralign, 128)],
                    buf_v.at[:, :, pl.ds(j * 128, 128)],
                    sem,
                ))
            for cp in cps:
                cp.wait()

            low = vec & jnp.int32(127)
            pos = lane * 128 + low
            for ti in range(2):
                for cc in range(8):
                    vals = plsc.load_gather(
                        buf_v, [lane * 0 + ti, lane * 0 + cc, pos]
                    )
                    rows_v[ti, cc, pl.ds(c0, 16)] = vals
            return _

        lax.fori_loop(0, n_chunks, chunk_body, 0)

        for ti in range(2):
            pltpu.sync_copy(
                rows_v.at[ti], out_hbm.at[ti, :, pl.ds(base, b_per_w)]
            )

    return gather_kernel


def kernel(g, h, table):
    B = h.shape[0]
    V = table.shape[0]
    info = plsc.get_sparse_core_info()
    NW = info.num_cores * info.num_subcores
    idx = h.astype(jnp.int32).reshape(NW, B // NW)
    tab3 = table.T.reshape(2, 8, V)
    out3 = _build(B, V)(idx, tab3)
    return out3.reshape(16, B).T
